# Initial kernel scaffold; baseline (speedup 1.0000x reference)
import jax
import jax.numpy as jnp
from jax.experimental import pallas as pl


def kernel(neighbors, emb_features, W):
    raise NotImplementedError("write your pallas kernel here")


# speedup vs baseline: 1.1381x; 1.1381x over previous
import functools

import jax
import jax.numpy as jnp
from jax import lax
from jax.experimental import pallas as pl
from jax.experimental.pallas import tpu as pltpu
from jax.experimental.pallas import tpu_sc as plsc

_N = 10000
_K = 32
_D = 128
_NW = 32
_NODES_PER_W = 320
_N_PAD = _NW * _NODES_PER_W
_NODES_PER_BATCH = 4
_ROWS_PER_BATCH = _NODES_PER_BATCH * _K
_N_BATCH = _NODES_PER_W // _NODES_PER_BATCH
_LANES = 16


def _sc_gather_sum(idx3, emb):
    mesh = plsc.VectorSubcoreMesh(core_axis_name="c", subcore_axis_name="s")

    @functools.partial(
        pl.kernel,
        mesh=mesh,
        out_type=jax.ShapeDtypeStruct((_NW, _NODES_PER_W, _D), jnp.float32),
        scratch_types=[
            pltpu.VMEM((_N_BATCH, _ROWS_PER_BATCH), jnp.int32),
            pltpu.VMEM((_ROWS_PER_BATCH, _D), jnp.float32),
            pltpu.VMEM((_NODES_PER_W, _D), jnp.float32),
            pltpu.SemaphoreType.DMA,
        ],
    )
    def k(idx_hbm, emb_hbm, out_hbm, idx_v, buf_v, acc_v, sem):
        wid = lax.axis_index("s") * 2 + lax.axis_index("c")
        pltpu.sync_copy(idx_hbm.at[wid], idx_v)

        @pl.loop(0, _N_BATCH)
        def _(b):
            pltpu.async_copy(emb_hbm.at[idx_v.at[b]], buf_v, sem).wait()
            for j in range(_NODES_PER_BATCH):
                node = b * _NODES_PER_BATCH + j
                for c in range(_D // _LANES):

                    def body(r, acc, j=j, c=c):
                        return acc + buf_v[j * _K + r, pl.ds(c * _LANES, _LANES)]

                    acc0 = buf_v[j * _K, pl.ds(c * _LANES, _LANES)]
                    accv = lax.fori_loop(1, _K, body, acc0)
                    acc_v[node, pl.ds(c * _LANES, _LANES)] = accv

        pltpu.sync_copy(acc_v, out_hbm.at[wid])

    return k(idx3, emb)


def _tc_linear_norm(sums, wt):
    blk = 1024

    def body(x_ref, w_ref, o_ref):
        x = x_ref[...] * (1.0 / _K)
        y = jnp.dot(x, w_ref[...], preferred_element_type=jnp.float32)
        y = jnp.maximum(y, 0.0)
        n = jnp.sqrt(jnp.sum(y * y, axis=1, keepdims=True))
        o_ref[...] = y / jnp.maximum(n, 1e-12)

    return pl.pallas_call(
        body,
        grid=(_N_PAD // blk,),
        in_specs=[
            pl.BlockSpec((blk, _D), lambda i: (i, 0)),
            pl.BlockSpec((_D, _D), lambda i: (0, 0)),
        ],
        out_specs=pl.BlockSpec((blk, _D), lambda i: (i, 0)),
        out_shape=jax.ShapeDtypeStruct((_N_PAD, _D), jnp.float32),
    )(sums, wt)


def kernel(neighbors, emb_features, W):
    nb = neighbors.astype(jnp.int32).reshape(-1)
    nb = jnp.concatenate([nb, jnp.zeros((_N_PAD * _K - _N * _K,), jnp.int32)])
    idx3 = nb.reshape(_NW, _N_BATCH, _ROWS_PER_BATCH)
    sums = _sc_gather_sum(idx3, emb_features).reshape(_N_PAD, _D)
    out = _tc_linear_norm(sums, W.T)
    return out[:_N]
